# Initial kernel scaffold; baseline (speedup 1.0000x reference)
import jax
import jax.numpy as jnp
from jax.experimental import pallas as pl


def kernel(x, xyz, Wq, bq, Wk, bk, Wv, bv, Wo, bo, Wb1, bb1, Wb2, bb2):
    raise NotImplementedError("write your pallas kernel here")


# speedup vs baseline: 7.6728x; 7.6728x over previous
import functools

import jax
import jax.numpy as jnp
from jax import lax
from jax.experimental import pallas as pl
from jax.experimental.pallas import tpu as pltpu
from jax.experimental.pallas import tpu_sc as plsc

DIM = 512
H = 16
K = 32
HD = DIM // H
BIG = 3.0e38


def _qkv_body(x_ref, wq_ref, bq_ref, wk_ref, bk_ref, wv_ref, bv_ref,
              q_ref, k_ref, v_ref):
    xb = x_ref[...]
    q_ref[...] = jnp.dot(xb, wq_ref[...],
                         preferred_element_type=jnp.float32) + bq_ref[...]
    k_ref[...] = jnp.dot(xb, wk_ref[...],
                         preferred_element_type=jnp.float32) + bk_ref[...]
    v_ref[...] = jnp.dot(xb, wv_ref[...],
                         preferred_element_type=jnp.float32) + bv_ref[...]


def _qkv_proj(x2d, wq, bq, wk, bk, wv, bv, blk):
    n = x2d.shape[0]
    bs_w = pl.BlockSpec((DIM, DIM), lambda i: (0, 0))
    bs_b = pl.BlockSpec((1, DIM), lambda i: (0, 0))
    bs_x = pl.BlockSpec((blk, DIM), lambda i: (i, 0))
    out_sd = jax.ShapeDtypeStruct((n, DIM), jnp.float32)
    return pl.pallas_call(
        _qkv_body,
        grid=(n // blk,),
        in_specs=[bs_x, bs_w, bs_b, bs_w, bs_b, bs_w, bs_b],
        out_specs=[bs_x, bs_x, bs_x],
        out_shape=[out_sd, out_sd, out_sd],
    )(x2d, wq, bq.reshape(1, DIM), wk, bk.reshape(1, DIM),
      wv, bv.reshape(1, DIM))


def _sc_knn_rel(xyzx, xyzy, xyzz, xbx, xby, xbz, rows, n):
    nw = 32
    per_w = rows // nw
    qchunk = 64
    ngrp = n // 16
    cap = n + 16
    mesh = plsc.VectorSubcoreMesh(core_axis_name="c", subcore_axis_name="s")

    @functools.partial(
        pl.kernel,
        mesh=mesh,
        compiler_params=pltpu.CompilerParams(needs_layout_passes=False),
        out_type=[
            jax.ShapeDtypeStruct((rows * K,), jnp.int32),
            jax.ShapeDtypeStruct((rows * K * 16,), jnp.float32),
        ],
        scratch_types=[
            pltpu.VMEM((rows,), jnp.float32),
            pltpu.VMEM((rows,), jnp.float32),
            pltpu.VMEM((rows,), jnp.float32),
            pltpu.VMEM((rows,), jnp.float32),
            pltpu.VMEM((rows,), jnp.float32),
            pltpu.VMEM((rows,), jnp.float32),
            pltpu.VMEM((rows,), jnp.float32),
            pltpu.VMEM((n,), jnp.float32),
            pltpu.VMEM((cap,), jnp.float32),
            pltpu.VMEM((cap,), jnp.int32),
            pltpu.VMEM((qchunk * K,), jnp.int32),
            pltpu.VMEM((qchunk * K * 16,), jnp.float32),
        ],
    )
    def k(xx_hbm, xy_hbm, xz_hbm, bx_hbm, by_hbm, bz_hbm, idx_hbm, rel_hbm,
          xx, xy, xz, bx, by, bz, n2, d2r, cval, cidx, idx_v, nb_v):
        cid = lax.axis_index("c")
        sid = lax.axis_index("s")
        wid = sid * 2 + cid
        base = pl.multiple_of(wid * per_w, 8)
        boff = pl.multiple_of((wid // 16) * n, 8)
        pltpu.sync_copy(xx_hbm, xx)
        pltpu.sync_copy(xy_hbm, xy)
        pltpu.sync_copy(xz_hbm, xz)
        pltpu.sync_copy(bx_hbm, bx)
        pltpu.sync_copy(by_hbm, by)
        pltpu.sync_copy(bz_hbm, bz)
        iota16 = lax.iota(jnp.int32, 16)
        zeros16 = jnp.zeros((16,), jnp.float32)
        bigv = jnp.full((16,), BIG, jnp.float32)
        ibig = jnp.full((16,), jnp.int32(2**31 - 1), jnp.int32)

        def n2_body(g, _):
            vx = xx[pl.ds(g * 16, 16)]
            vy = xy[pl.ds(g * 16, 16)]
            vz = xz[pl.ds(g * 16, 16)]
            n2[pl.ds(g * 16, 16)] = vx * vx + vy * vy + vz * vz
            return 0

        lax.fori_loop(0, rows // 16, n2_body, 0)

        def z_body(z, _):
            nb_v[pl.ds(z * 16, 16)] = zeros16
            return 0

        lax.fori_loop(0, qchunk * K, z_body, 0)

        def q_body(qi2, _):
            ci = qi2 // qchunk
            qi = qi2 - ci * qchunk
            r = base + qi2
            rsp = jnp.full((16,), r, jnp.int32)
            qxv = plsc.load_gather(xx, [rsp])
            qyv = plsc.load_gather(xy, [rsp])
            qzv = plsc.load_gather(xz, [rsp])
            qbx = plsc.load_gather(bx, [rsp])
            qby = plsc.load_gather(by, [rsp])
            qbz = plsc.load_gather(bz, [rsp])
            qnv = plsc.load_gather(n2, [rsp])

            def p1(c, carry):
                m1, m2 = carry
                o = pl.multiple_of(boff + c * 16, 8)
                s = (bx[pl.ds(o, 16)] * qbx + by[pl.ds(o, 16)] * qby
                     + bz[pl.ds(o, 16)] * qbz)
                d = jnp.maximum(qnv + n2[pl.ds(o, 16)] - 2.0 * s, 0.0)
                d2r[pl.ds(c * 16, 16)] = d
                upd = d < m1
                m2 = jnp.where(upd, m1, jnp.minimum(m2, d))
                m1 = jnp.minimum(m1, d)
                return m1, m2

            _, m2 = lax.fori_loop(0, ngrp, p1, (bigv, bigv))
            thr = jnp.full((16,), jnp.max(m2), jnp.float32)

            def p2(c, off):
                v = d2r[pl.ds(c * 16, 16)]
                msk = v <= thr
                plsc.store_compressed(cval.at[pl.ds(off, 16)], v, mask=msk)
                plsc.store_compressed(cidx.at[pl.ds(off, 16)],
                                      boff + c * 16 + iota16, mask=msk)
                pc = plsc.all_reduce_population_count(msk)
                return off + pc[0]

            cnt = lax.fori_loop(0, ngrp, p2, jnp.int32(0))
            cval[pl.ds(cnt, 16)] = bigv
            nv = (cnt + 15) // 16

            def ext(t, sel):
                sel_lo, sel_hi = sel

                def scan(c2, carry):
                    m, p = carry
                    v = cval[pl.ds(c2 * 16, 16)]
                    upd = v < m
                    m = jnp.where(upd, v, m)
                    p = jnp.where(upd, c2 * 16 + iota16, p)
                    return m, p

                m, p = lax.fori_loop(0, nv, scan,
                                     (bigv, jnp.zeros((16,), jnp.int32)))
                gm = jnp.min(m)
                gp = jnp.min(jnp.where(m == gm, p, ibig))
                gpv = jnp.full((16,), gp, jnp.int32)
                gidxv = plsc.load_gather(cidx, [gpv])
                plsc.store_scatter(cval, [gpv], bigv)
                sel_lo = jnp.where(iota16 == t, gidxv, sel_lo)
                sel_hi = jnp.where(iota16 == t - 16, gidxv, sel_hi)
                return sel_lo, sel_hi

            sel_lo, sel_hi = lax.fori_loop(
                0, K, ext,
                (jnp.zeros((16,), jnp.int32), jnp.zeros((16,), jnp.int32)))
            idx_v[pl.ds(qi * K, 16)] = sel_lo
            idx_v[pl.ds(qi * K + 16, 16)] = sel_hi

            for half, sel in ((0, sel_lo), (1, sel_hi)):
                offs = qi * K * 16 + half * 256 + iota16 * 16
                plsc.store_scatter(nb_v, [offs],
                                   qxv - plsc.load_gather(xx, [sel]))
                plsc.store_scatter(nb_v, [offs + 1],
                                   qyv - plsc.load_gather(xy, [sel]))
                plsc.store_scatter(nb_v, [offs + 2],
                                   qzv - plsc.load_gather(xz, [sel]))

            @pl.when(qi == qchunk - 1)
            def _flush():
                row0 = pl.multiple_of(base + ci * qchunk, 8)
                pltpu.sync_copy(
                    idx_v, idx_hbm.at[pl.ds(row0 * K, qchunk * K)])
                pltpu.sync_copy(
                    nb_v, rel_hbm.at[pl.ds(row0 * K * 16, qchunk * K * 16)])

            return 0

        lax.fori_loop(0, per_w, q_body, 0)

    return k(xyzx, xyzy, xyzz, xbx, xby, xbz)


def _bias_body(rel_ref, w1_ref, b1_ref, w2_ref, b2_ref, out_ref):
    h1 = jnp.dot(rel_ref[...], w1_ref[...],
                 preferred_element_type=jnp.float32) + b1_ref[...]
    h1 = jnp.maximum(h1, 0.0)
    out_ref[...] = jnp.dot(h1, w2_ref[...],
                           preferred_element_type=jnp.float32) + b2_ref[...]


def _bias_mlp(rel16, w1p, b1, w2t, b2, blk):
    rows = rel16.shape[0]
    return pl.pallas_call(
        _bias_body,
        grid=(rows // blk,),
        in_specs=[
            pl.BlockSpec((blk, 16), lambda i: (i, 0)),
            pl.BlockSpec((16, 32), lambda i: (0, 0)),
            pl.BlockSpec((1, 32), lambda i: (0, 0)),
            pl.BlockSpec((32, 16), lambda i: (0, 0)),
            pl.BlockSpec((1, 16), lambda i: (0, 0)),
        ],
        out_specs=pl.BlockSpec((blk, 16), lambda i: (i, 0)),
        out_shape=jax.ShapeDtypeStruct((rows, 16), jnp.float32),
    )(rel16, w1p, b1.reshape(1, 32), w2t, b2.reshape(1, 16))


def _sc_attend(qT, kT, vT, bias3, idx2d, rows):
    nw = 32
    per_w = rows // nw
    scale = float(1.0 / (HD ** 0.5))
    mesh = plsc.VectorSubcoreMesh(core_axis_name="c", subcore_axis_name="s")

    @functools.partial(
        pl.kernel,
        mesh=mesh,
        compiler_params=pltpu.CompilerParams(needs_layout_passes=False),
        out_type=jax.ShapeDtypeStruct((rows, DIM), jnp.float32),
        scratch_types=[
            pltpu.VMEM((per_w * K,), jnp.int32),
            pltpu.VMEM((K, DIM), jnp.float32),
            pltpu.VMEM((K, DIM), jnp.float32),
            pltpu.VMEM((DIM,), jnp.float32),
            pltpu.VMEM((K * H,), jnp.float32),
            pltpu.VMEM((DIM,), jnp.float32),
            pltpu.SemaphoreType.DMA,
            pltpu.SemaphoreType.DMA,
        ],
    )
    def k(q_hbm, k_hbm, v_hbm, bias_hbm, idx_hbm, out_hbm,
          idx_v, kb, vb, qb, sb, ob, semk, semv):
        cid = lax.axis_index("c")
        sid = lax.axis_index("s")
        wid = sid * 2 + cid
        base = pl.multiple_of(wid * per_w, 8)
        pltpu.sync_copy(idx_hbm.at[pl.ds(base * K, per_w * K)], idx_v)

        def q_body(qi, _):
            r = base + qi
            i0 = pl.multiple_of(qi * K, 8)
            ck = pltpu.async_copy(k_hbm.at[idx_v.at[pl.ds(i0, K)]], kb, semk)
            cv = pltpu.async_copy(v_hbm.at[idx_v.at[pl.ds(i0, K)]], vb, semv)
            pltpu.sync_copy(q_hbm.at[r], qb)
            pltpu.sync_copy(bias_hbm.at[r], sb)
            ck.wait()

            def sc_body(d, accs):
                qv = qb[pl.ds(d * H, H)]
                return tuple(accs[j] + qv * kb[j, pl.ds(d * H, H)]
                             for j in range(K))

            accs = lax.fori_loop(
                0, HD, sc_body,
                tuple(jnp.zeros((H,), jnp.float32) for _ in range(K)))
            s = [accs[j] * scale + sb[pl.ds(j * H, H)] for j in range(K)]

            t = list(s)
            while len(t) > 1:
                t = [jnp.maximum(t[2 * i], t[2 * i + 1])
                     for i in range(len(t) // 2)]
            m = t[0]
            den = jnp.zeros((H,), jnp.float32)
            for j in range(K):
                e = jnp.exp(s[j] - m)
                sb[pl.ds(j * H, H)] = e
                den = den + e
            rden = 1.0 / den

            cv.wait()

            def out_body(j, oaccs):
                a = sb[pl.ds(j * H, H)]
                return tuple(oaccs[d] + a * vb[j, pl.ds(d * H, H)]
                             for d in range(HD))

            oaccs = lax.fori_loop(
                0, K, out_body,
                tuple(jnp.zeros((H,), jnp.float32) for _ in range(HD)))
            for d in range(HD):
                ob[pl.ds(d * H, H)] = oaccs[d] * rden
            pltpu.sync_copy(ob, out_hbm.at[r])
            return 0

        lax.fori_loop(0, per_w, q_body, 0)

    return k(qT, kT, vT, bias3, idx2d)


def _proj_body(x_ref, w_ref, b_ref, o_ref):
    o_ref[...] = jnp.dot(x_ref[...], w_ref[...],
                         preferred_element_type=jnp.float32) + b_ref[...]


def _out_proj(x2d, w, b, blk):
    n = x2d.shape[0]
    return pl.pallas_call(
        _proj_body,
        grid=(n // blk,),
        in_specs=[
            pl.BlockSpec((blk, DIM), lambda i: (i, 0)),
            pl.BlockSpec((DIM, DIM), lambda i: (0, 0)),
            pl.BlockSpec((1, DIM), lambda i: (0, 0)),
        ],
        out_specs=pl.BlockSpec((blk, DIM), lambda i: (i, 0)),
        out_shape=jax.ShapeDtypeStruct((n, DIM), jnp.float32),
    )(x2d, w, b.reshape(1, DIM))


def kernel(x, xyz, Wq, bq, Wk, bk, Wv, bv, Wo, bo, Wb1, bb1, Wb2, bb2):
    B, N, C = x.shape
    rows = B * N

    a = jnp.arange(DIM)
    p = (a % H) * HD + a // H

    wq_t = Wq[p, :].T
    wk_t = Wk[p, :].T
    wv_t = Wv[p, :].T
    wo_t = Wo[:, p].T

    x2d = x.reshape(rows, DIM)
    qT, kT, vT = _qkv_proj(x2d, wq_t, bq[p], wk_t, bk[p], wv_t, bv[p],
                           blk=256)

    xyz2d = xyz.reshape(rows, 3)
    u = lax.bitcast_convert_type(xyz2d, jnp.uint32)
    u = (u + jnp.uint32(0x7FFF) + ((u >> 16) & jnp.uint32(1))) \
        & jnp.uint32(0xFFFF0000)
    xyzb = lax.bitcast_convert_type(u, jnp.float32)
    idx_flat, rel1 = _sc_knn_rel(xyz2d[:, 0], xyz2d[:, 1], xyz2d[:, 2],
                                 xyzb[:, 0], xyzb[:, 1], xyzb[:, 2],
                                 rows, N)
    rel2 = rel1.reshape(rows * K, 16)

    w1p = jnp.concatenate(
        [Wb1, jnp.zeros((32, 13), jnp.float32)], axis=1).T
    bias2 = _bias_mlp(rel2, w1p, bb1, Wb2.T, bb2, blk=4096)

    outT = _sc_attend(qT, kT, vT, bias2.reshape(rows, K * H),
                      idx_flat, rows)

    y = _out_proj(outT, wo_t, bo, blk=256)
    return y.reshape(B, N, C)


# speedup vs baseline: 8.2231x; 1.0717x over previous
import functools

import jax
import jax.numpy as jnp
from jax import lax
from jax.experimental import pallas as pl
from jax.experimental.pallas import tpu as pltpu
from jax.experimental.pallas import tpu_sc as plsc

DIM = 512
H = 16
K = 32
HD = DIM // H
BIG = 3.0e38


def _qkv_body(x_ref, wq_ref, bq_ref, wk_ref, bk_ref, wv_ref, bv_ref,
              q_ref, k_ref, v_ref):
    xb = x_ref[...]
    q_ref[...] = jnp.dot(xb, wq_ref[...],
                         preferred_element_type=jnp.float32) + bq_ref[...]
    k_ref[...] = jnp.dot(xb, wk_ref[...],
                         preferred_element_type=jnp.float32) + bk_ref[...]
    v_ref[...] = jnp.dot(xb, wv_ref[...],
                         preferred_element_type=jnp.float32) + bv_ref[...]


def _qkv_proj(x2d, wq, bq, wk, bk, wv, bv, blk):
    n = x2d.shape[0]
    bs_w = pl.BlockSpec((DIM, DIM), lambda i: (0, 0))
    bs_b = pl.BlockSpec((1, DIM), lambda i: (0, 0))
    bs_x = pl.BlockSpec((blk, DIM), lambda i: (i, 0))
    out_sd = jax.ShapeDtypeStruct((n, DIM), jnp.float32)
    return pl.pallas_call(
        _qkv_body,
        grid=(n // blk,),
        in_specs=[bs_x, bs_w, bs_b, bs_w, bs_b, bs_w, bs_b],
        out_specs=[bs_x, bs_x, bs_x],
        out_shape=[out_sd, out_sd, out_sd],
    )(x2d, wq, bq.reshape(1, DIM), wk, bk.reshape(1, DIM),
      wv, bv.reshape(1, DIM))


def _sc_knn_rel(xyzx, xyzy, xyzz, xbx, xby, xbz, rows, n):
    nw = 32
    per_w = rows // nw
    qchunk = 64
    ngrp = n // 16
    cap = n + 16
    mesh = plsc.VectorSubcoreMesh(core_axis_name="c", subcore_axis_name="s")

    @functools.partial(
        pl.kernel,
        mesh=mesh,
        compiler_params=pltpu.CompilerParams(needs_layout_passes=False),
        out_type=[
            jax.ShapeDtypeStruct((rows * K,), jnp.int32),
            jax.ShapeDtypeStruct((rows * K * 16,), jnp.float32),
        ],
        scratch_types=[
            pltpu.VMEM((rows,), jnp.float32),
            pltpu.VMEM((rows,), jnp.float32),
            pltpu.VMEM((rows,), jnp.float32),
            pltpu.VMEM((rows,), jnp.float32),
            pltpu.VMEM((rows,), jnp.float32),
            pltpu.VMEM((rows,), jnp.float32),
            pltpu.VMEM((rows,), jnp.float32),
            pltpu.VMEM((n,), jnp.float32),
            pltpu.VMEM((cap,), jnp.float32),
            pltpu.VMEM((cap,), jnp.int32),
            pltpu.VMEM((qchunk * K,), jnp.int32),
            pltpu.VMEM((qchunk * K * 16,), jnp.float32),
        ],
    )
    def k(xx_hbm, xy_hbm, xz_hbm, bx_hbm, by_hbm, bz_hbm, idx_hbm, rel_hbm,
          xx, xy, xz, bx, by, bz, n2, d2r, cval, cidx, idx_v, nb_v):
        cid = lax.axis_index("c")
        sid = lax.axis_index("s")
        wid = sid * 2 + cid
        base = pl.multiple_of(wid * per_w, 8)
        boff = pl.multiple_of((wid // 16) * n, 8)
        pltpu.sync_copy(xx_hbm, xx)
        pltpu.sync_copy(xy_hbm, xy)
        pltpu.sync_copy(xz_hbm, xz)
        pltpu.sync_copy(bx_hbm, bx)
        pltpu.sync_copy(by_hbm, by)
        pltpu.sync_copy(bz_hbm, bz)
        iota16 = lax.iota(jnp.int32, 16)
        zeros16 = jnp.zeros((16,), jnp.float32)
        bigv = jnp.full((16,), BIG, jnp.float32)
        ibig = jnp.full((16,), jnp.int32(2**31 - 1), jnp.int32)

        def n2_body(g, _):
            vx = xx[pl.ds(g * 16, 16)]
            vy = xy[pl.ds(g * 16, 16)]
            vz = xz[pl.ds(g * 16, 16)]
            n2[pl.ds(g * 16, 16)] = vx * vx + vy * vy + vz * vz
            return 0

        lax.fori_loop(0, rows // 16, n2_body, 0)

        def z_body(z, _):
            nb_v[pl.ds(z * 16, 16)] = zeros16
            return 0

        lax.fori_loop(0, qchunk * K, z_body, 0)

        def q_body(qi2, _):
            ci = qi2 // qchunk
            qi = qi2 - ci * qchunk
            r = base + qi2
            rsp = jnp.full((16,), r, jnp.int32)
            qxv = plsc.load_gather(xx, [rsp])
            qyv = plsc.load_gather(xy, [rsp])
            qzv = plsc.load_gather(xz, [rsp])
            qbx = plsc.load_gather(bx, [rsp])
            qby = plsc.load_gather(by, [rsp])
            qbz = plsc.load_gather(bz, [rsp])
            qnv = plsc.load_gather(n2, [rsp])

            def p1(c, carry):
                m1, m2 = carry
                o = pl.multiple_of(boff + c * 16, 8)
                s = (bx[pl.ds(o, 16)] * qbx + by[pl.ds(o, 16)] * qby
                     + bz[pl.ds(o, 16)] * qbz)
                d = jnp.maximum(qnv + n2[pl.ds(o, 16)] - 2.0 * s, 0.0)
                d2r[pl.ds(c * 16, 16)] = d
                upd = d < m1
                m2 = jnp.where(upd, m1, jnp.minimum(m2, d))
                m1 = jnp.minimum(m1, d)
                return m1, m2

            _, m2 = lax.fori_loop(0, ngrp, p1, (bigv, bigv))
            thr = jnp.full((16,), jnp.max(m2), jnp.float32)

            def p2(c, off_v):
                v = d2r[pl.ds(c * 16, 16)]
                msk = v <= thr
                cs = plsc.cumsum(jnp.where(msk, 1, 0).astype(jnp.int32))
                pos = off_v + cs - 1
                plsc.store_scatter(cval, [pos], v, mask=msk)
                plsc.store_scatter(cidx, [pos], boff + c * 16 + iota16,
                                   mask=msk)
                pc = plsc.all_reduce_population_count(msk)
                return off_v + pc

            off_v = lax.fori_loop(0, ngrp, p2, jnp.zeros((16,), jnp.int32))
            cnt = off_v[0]
            cval[pl.ds(cnt, 16)] = bigv
            nv = (cnt + 15) // 16

            def ext(c2, carry):
                ak, av, bk, bv = carry
                ck = cval[pl.ds(c2 * 16, 16)]
                cv = cidx[pl.ds(c2 * 16, 16)]
                ck, cv = plsc.sort_key_val(ck, cv)
                rk = lax.rev(ck, (0,))
                rv = lax.rev(cv, (0,))
                mm = bk <= rk
                lk = jnp.where(mm, bk, rk)
                lv = jnp.where(mm, bv, rv)
                lk, lv = plsc.sort_key_val(lk, lv)
                rk2 = lax.rev(lk, (0,))
                rv2 = lax.rev(lv, (0,))
                m2 = ak <= rk2
                nak = jnp.where(m2, ak, rk2)
                nav = jnp.where(m2, av, rv2)
                hbk = jnp.where(m2, rk2, ak)
                hbv = jnp.where(m2, rv2, av)
                ak, av = plsc.sort_key_val(nak, nav)
                bk, bv = plsc.sort_key_val(hbk, hbv)
                return ak, av, bk, bv

            zi = jnp.zeros((16,), jnp.int32)
            _, sel_lo, _, sel_hi = lax.fori_loop(
                0, nv, ext, (bigv, zi, bigv, zi))
            idx_v[pl.ds(qi * K, 16)] = sel_lo
            idx_v[pl.ds(qi * K + 16, 16)] = sel_hi

            for half, sel in ((0, sel_lo), (1, sel_hi)):
                offs = qi * K * 16 + half * 256 + iota16 * 16
                plsc.store_scatter(nb_v, [offs],
                                   qxv - plsc.load_gather(xx, [sel]))
                plsc.store_scatter(nb_v, [offs + 1],
                                   qyv - plsc.load_gather(xy, [sel]))
                plsc.store_scatter(nb_v, [offs + 2],
                                   qzv - plsc.load_gather(xz, [sel]))

            @pl.when(qi == qchunk - 1)
            def _flush():
                row0 = pl.multiple_of(base + ci * qchunk, 8)
                pltpu.sync_copy(
                    idx_v, idx_hbm.at[pl.ds(row0 * K, qchunk * K)])
                pltpu.sync_copy(
                    nb_v, rel_hbm.at[pl.ds(row0 * K * 16, qchunk * K * 16)])

            return 0

        lax.fori_loop(0, per_w, q_body, 0)

    return k(xyzx, xyzy, xyzz, xbx, xby, xbz)


def _bias_body(rel_ref, w1_ref, b1_ref, w2_ref, b2_ref, out_ref):
    h1 = jnp.dot(rel_ref[...], w1_ref[...],
                 preferred_element_type=jnp.float32) + b1_ref[...]
    h1 = jnp.maximum(h1, 0.0)
    out_ref[...] = jnp.dot(h1, w2_ref[...],
                           preferred_element_type=jnp.float32) + b2_ref[...]


def _bias_mlp(rel16, w1p, b1, w2t, b2, blk):
    rows = rel16.shape[0]
    return pl.pallas_call(
        _bias_body,
        grid=(rows // blk,),
        in_specs=[
            pl.BlockSpec((blk, 16), lambda i: (i, 0)),
            pl.BlockSpec((16, 32), lambda i: (0, 0)),
            pl.BlockSpec((1, 32), lambda i: (0, 0)),
            pl.BlockSpec((32, 16), lambda i: (0, 0)),
            pl.BlockSpec((1, 16), lambda i: (0, 0)),
        ],
        out_specs=pl.BlockSpec((blk, 16), lambda i: (i, 0)),
        out_shape=jax.ShapeDtypeStruct((rows, 16), jnp.float32),
    )(rel16, w1p, b1.reshape(1, 32), w2t, b2.reshape(1, 16))


def _sc_attend(qT, kT, vT, bias3, idx2d, rows):
    nw = 32
    per_w = rows // nw
    scale = float(1.0 / (HD ** 0.5))
    mesh = plsc.VectorSubcoreMesh(core_axis_name="c", subcore_axis_name="s")

    @functools.partial(
        pl.kernel,
        mesh=mesh,
        compiler_params=pltpu.CompilerParams(needs_layout_passes=False),
        out_type=jax.ShapeDtypeStruct((rows, DIM), jnp.float32),
        scratch_types=[
            pltpu.VMEM((per_w * K,), jnp.int32),
            pltpu.VMEM((K, DIM), jnp.float32),
            pltpu.VMEM((K, DIM), jnp.float32),
            pltpu.VMEM((K, DIM), jnp.float32),
            pltpu.VMEM((K, DIM), jnp.float32),
            pltpu.VMEM((DIM,), jnp.float32),
            pltpu.VMEM((K * H,), jnp.float32),
            pltpu.VMEM((DIM,), jnp.float32),
            pltpu.SemaphoreType.DMA,
            pltpu.SemaphoreType.DMA,
            pltpu.SemaphoreType.DMA,
            pltpu.SemaphoreType.DMA,
        ],
    )
    def k(q_hbm, k_hbm, v_hbm, bias_hbm, idx_hbm, out_hbm,
          idx_v, kb0, vb0, kb1, vb1, qb, sb, ob, semk0, semv0, semk1, semv1):
        cid = lax.axis_index("c")
        sid = lax.axis_index("s")
        wid = sid * 2 + cid
        base = pl.multiple_of(wid * per_w, 8)
        pltpu.sync_copy(idx_hbm.at[pl.ds(base * K, per_w * K)], idx_v)

        def attend_one(r, kb, vb, ck, cv):
            pltpu.sync_copy(q_hbm.at[r], qb)
            pltpu.sync_copy(bias_hbm.at[r], sb)
            ck.wait()

            def sc_body(d, accs):
                qv = qb[pl.ds(d * H, H)]
                return tuple(accs[j] + qv * kb[j, pl.ds(d * H, H)]
                             for j in range(K))

            accs = lax.fori_loop(
                0, HD, sc_body,
                tuple(jnp.zeros((H,), jnp.float32) for _ in range(K)))
            s = [accs[j] * scale + sb[pl.ds(j * H, H)] for j in range(K)]

            t = list(s)
            while len(t) > 1:
                t = [jnp.maximum(t[2 * i], t[2 * i + 1])
                     for i in range(len(t) // 2)]
            m = t[0]
            den = jnp.zeros((H,), jnp.float32)
            for j in range(K):
                e = jnp.exp(s[j] - m)
                sb[pl.ds(j * H, H)] = e
                den = den + e
            rden = 1.0 / den

            cv.wait()

            def out_body(j, oaccs):
                a = sb[pl.ds(j * H, H)]
                return tuple(oaccs[d] + a * vb[j, pl.ds(d * H, H)]
                             for d in range(HD))

            oaccs = lax.fori_loop(
                0, K, out_body,
                tuple(jnp.zeros((H,), jnp.float32) for _ in range(HD)))
            for d in range(HD):
                ob[pl.ds(d * H, H)] = oaccs[d] * rden
            pltpu.sync_copy(ob, out_hbm.at[r])

        def q_body(qp, _):
            qi = qp * 2
            i0 = pl.multiple_of(qi * K, 8)
            i1 = pl.multiple_of(qi * K + K, 8)
            ck0 = pltpu.async_copy(k_hbm.at[idx_v.at[pl.ds(i0, K)]], kb0,
                                   semk0)
            cv0 = pltpu.async_copy(v_hbm.at[idx_v.at[pl.ds(i0, K)]], vb0,
                                   semv0)
            ck1 = pltpu.async_copy(k_hbm.at[idx_v.at[pl.ds(i1, K)]], kb1,
                                   semk1)
            cv1 = pltpu.async_copy(v_hbm.at[idx_v.at[pl.ds(i1, K)]], vb1,
                                   semv1)
            attend_one(base + qi, kb0, vb0, ck0, cv0)
            attend_one(base + qi + 1, kb1, vb1, ck1, cv1)
            return 0

        lax.fori_loop(0, per_w // 2, q_body, 0)

    return k(qT, kT, vT, bias3, idx2d)


def _proj_body(x_ref, w_ref, b_ref, o_ref):
    o_ref[...] = jnp.dot(x_ref[...], w_ref[...],
                         preferred_element_type=jnp.float32) + b_ref[...]


def _out_proj(x2d, w, b, blk):
    n = x2d.shape[0]
    return pl.pallas_call(
        _proj_body,
        grid=(n // blk,),
        in_specs=[
            pl.BlockSpec((blk, DIM), lambda i: (i, 0)),
            pl.BlockSpec((DIM, DIM), lambda i: (0, 0)),
            pl.BlockSpec((1, DIM), lambda i: (0, 0)),
        ],
        out_specs=pl.BlockSpec((blk, DIM), lambda i: (i, 0)),
        out_shape=jax.ShapeDtypeStruct((n, DIM), jnp.float32),
    )(x2d, w, b.reshape(1, DIM))


def kernel(x, xyz, Wq, bq, Wk, bk, Wv, bv, Wo, bo, Wb1, bb1, Wb2, bb2):
    B, N, C = x.shape
    rows = B * N

    a = jnp.arange(DIM)
    p = (a % H) * HD + a // H

    wq_t = Wq[p, :].T
    wk_t = Wk[p, :].T
    wv_t = Wv[p, :].T
    wo_t = Wo[:, p].T

    x2d = x.reshape(rows, DIM)
    qT, kT, vT = _qkv_proj(x2d, wq_t, bq[p], wk_t, bk[p], wv_t, bv[p],
                           blk=256)

    xyz2d = xyz.reshape(rows, 3)
    u = lax.bitcast_convert_type(xyz2d, jnp.uint32)
    u = (u + jnp.uint32(0x7FFF) + ((u >> 16) & jnp.uint32(1))) \
        & jnp.uint32(0xFFFF0000)
    xyzb = lax.bitcast_convert_type(u, jnp.float32)
    idx_flat, rel1 = _sc_knn_rel(xyz2d[:, 0], xyz2d[:, 1], xyz2d[:, 2],
                                 xyzb[:, 0], xyzb[:, 1], xyzb[:, 2],
                                 rows, N)
    rel2 = rel1.reshape(rows * K, 16)

    w1p = jnp.concatenate(
        [Wb1, jnp.zeros((32, 13), jnp.float32)], axis=1).T
    bias2 = _bias_mlp(rel2, w1p, bb1, Wb2.T, bb2, blk=4096)

    outT = _sc_attend(qT, kT, vT, bias2.reshape(rows, K * H),
                      idx_flat, rows)

    y = _out_proj(outT, wo_t, bo, blk=256)
    return y.reshape(B, N, C)


# speedup vs baseline: 8.6510x; 1.0520x over previous
import functools

import jax
import jax.numpy as jnp
from jax import lax
from jax.experimental import pallas as pl
from jax.experimental.pallas import tpu as pltpu
from jax.experimental.pallas import tpu_sc as plsc

DIM = 512
H = 16
K = 32
HD = DIM // H
BIG = 3.0e38


def _qkv_body(x_ref, wq_ref, bq_ref, wk_ref, bk_ref, wv_ref, bv_ref,
              q_ref, k_ref, v_ref):
    xb = x_ref[...]
    q_ref[...] = jnp.dot(xb, wq_ref[...],
                         preferred_element_type=jnp.float32) + bq_ref[...]
    k_ref[...] = jnp.dot(xb, wk_ref[...],
                         preferred_element_type=jnp.float32) + bk_ref[...]
    v_ref[...] = jnp.dot(xb, wv_ref[...],
                         preferred_element_type=jnp.float32) + bv_ref[...]


def _qkv_proj(x2d, wq, bq, wk, bk, wv, bv, blk):
    n = x2d.shape[0]
    bs_w = pl.BlockSpec((DIM, DIM), lambda i: (0, 0))
    bs_b = pl.BlockSpec((1, DIM), lambda i: (0, 0))
    bs_x = pl.BlockSpec((blk, DIM), lambda i: (i, 0))
    out_sd = jax.ShapeDtypeStruct((n, DIM), jnp.float32)
    return pl.pallas_call(
        _qkv_body,
        grid=(n // blk,),
        in_specs=[bs_x, bs_w, bs_b, bs_w, bs_b, bs_w, bs_b],
        out_specs=[bs_x, bs_x, bs_x],
        out_shape=[out_sd, out_sd, out_sd],
    )(x2d, wq, bq.reshape(1, DIM), wk, bk.reshape(1, DIM),
      wv, bv.reshape(1, DIM))


def _sc_knn_rel(xyzx, xyzy, xyzz, xbx, xby, xbz, rows, n):
    nw = 32
    per_w = rows // nw
    qchunk = 64
    ngrp = n // 16
    cap = n + 16
    mesh = plsc.VectorSubcoreMesh(core_axis_name="c", subcore_axis_name="s")

    @functools.partial(
        pl.kernel,
        mesh=mesh,
        compiler_params=pltpu.CompilerParams(needs_layout_passes=False),
        out_type=[
            jax.ShapeDtypeStruct((rows * K,), jnp.int32),
            jax.ShapeDtypeStruct((rows * K * 16,), jnp.float32),
        ],
        scratch_types=[
            pltpu.VMEM((rows,), jnp.float32),
            pltpu.VMEM((rows,), jnp.float32),
            pltpu.VMEM((rows,), jnp.float32),
            pltpu.VMEM((rows,), jnp.float32),
            pltpu.VMEM((rows,), jnp.float32),
            pltpu.VMEM((rows,), jnp.float32),
            pltpu.VMEM((rows,), jnp.float32),
            pltpu.VMEM((n,), jnp.float32),
            pltpu.VMEM((cap,), jnp.float32),
            pltpu.VMEM((cap,), jnp.int32),
            pltpu.VMEM((qchunk * K,), jnp.int32),
            pltpu.VMEM((qchunk * K * 16,), jnp.float32),
        ],
    )
    def k(xx_hbm, xy_hbm, xz_hbm, bx_hbm, by_hbm, bz_hbm, idx_hbm, rel_hbm,
          xx, xy, xz, bx, by, bz, n2, d2r, cval, cidx, idx_v, nb_v):
        cid = lax.axis_index("c")
        sid = lax.axis_index("s")
        wid = sid * 2 + cid
        base = pl.multiple_of(wid * per_w, 8)
        boff = pl.multiple_of((wid // 16) * n, 8)
        pltpu.sync_copy(xx_hbm, xx)
        pltpu.sync_copy(xy_hbm, xy)
        pltpu.sync_copy(xz_hbm, xz)
        pltpu.sync_copy(bx_hbm, bx)
        pltpu.sync_copy(by_hbm, by)
        pltpu.sync_copy(bz_hbm, bz)
        iota16 = lax.iota(jnp.int32, 16)
        zeros16 = jnp.zeros((16,), jnp.float32)
        bigv = jnp.full((16,), BIG, jnp.float32)
        ibig = jnp.full((16,), jnp.int32(2**31 - 1), jnp.int32)

        def n2_body(g, _):
            vx = xx[pl.ds(g * 16, 16)]
            vy = xy[pl.ds(g * 16, 16)]
            vz = xz[pl.ds(g * 16, 16)]
            n2[pl.ds(g * 16, 16)] = vx * vx + vy * vy + vz * vz
            return 0

        lax.fori_loop(0, rows // 16, n2_body, 0)

        def z_body(z, _):
            nb_v[pl.ds(z * 16, 16)] = zeros16
            return 0

        lax.fori_loop(0, qchunk * K, z_body, 0)

        def q_body(qi2, _):
            ci = qi2 // qchunk
            qi = qi2 - ci * qchunk
            r = base + qi2
            rsp = jnp.full((16,), r, jnp.int32)
            qxv = plsc.load_gather(xx, [rsp])
            qyv = plsc.load_gather(xy, [rsp])
            qzv = plsc.load_gather(xz, [rsp])
            qbx = plsc.load_gather(bx, [rsp])
            qby = plsc.load_gather(by, [rsp])
            qbz = plsc.load_gather(bz, [rsp])
            qnv = plsc.load_gather(n2, [rsp])

            def p1(c4, carry):
                m1, m2 = carry
                for u in range(4):
                    c = c4 * 4 + u
                    o = pl.multiple_of(boff + c * 16, 8)
                    s = (bx[pl.ds(o, 16)] * qbx + by[pl.ds(o, 16)] * qby
                         + bz[pl.ds(o, 16)] * qbz)
                    d = jnp.maximum(qnv + n2[pl.ds(o, 16)] - 2.0 * s, 0.0)
                    d2r[pl.ds(c * 16, 16)] = d
                    upd = d < m1
                    m2 = jnp.where(upd, m1, jnp.minimum(m2, d))
                    m1 = jnp.minimum(m1, d)
                return m1, m2

            _, m2 = lax.fori_loop(0, ngrp // 4, p1, (bigv, bigv))
            thr = jnp.full((16,), jnp.max(m2), jnp.float32)

            def p2(c4, off_v):
                for u in range(4):
                    c = c4 * 4 + u
                    v = d2r[pl.ds(c * 16, 16)]
                    msk = v <= thr
                    cs = plsc.cumsum(jnp.where(msk, 1, 0).astype(jnp.int32))
                    pos = off_v + cs - 1
                    plsc.store_scatter(cval, [pos], v, mask=msk)
                    plsc.store_scatter(cidx, [pos], boff + c * 16 + iota16,
                                       mask=msk)
                    pc = plsc.all_reduce_population_count(msk)
                    off_v = off_v + pc
                return off_v

            off_v = lax.fori_loop(0, ngrp // 4, p2,
                                  jnp.zeros((16,), jnp.int32))
            cnt = off_v[0]
            cval[pl.ds(cnt, 16)] = bigv
            nv = (cnt + 15) // 16

            def ext(c2, carry):
                ak, av, bk, bv = carry
                ck = cval[pl.ds(c2 * 16, 16)]
                cv = cidx[pl.ds(c2 * 16, 16)]
                ck, cv = plsc.sort_key_val(ck, cv)
                rk = lax.rev(ck, (0,))
                rv = lax.rev(cv, (0,))
                mm = bk <= rk
                lk = jnp.where(mm, bk, rk)
                lv = jnp.where(mm, bv, rv)
                lk, lv = plsc.sort_key_val(lk, lv)
                rk2 = lax.rev(lk, (0,))
                rv2 = lax.rev(lv, (0,))
                m2 = ak <= rk2
                nak = jnp.where(m2, ak, rk2)
                nav = jnp.where(m2, av, rv2)
                hbk = jnp.where(m2, rk2, ak)
                hbv = jnp.where(m2, rv2, av)
                ak, av = plsc.sort_key_val(nak, nav)
                bk, bv = plsc.sort_key_val(hbk, hbv)
                return ak, av, bk, bv

            zi = jnp.zeros((16,), jnp.int32)
            _, sel_lo, _, sel_hi = lax.fori_loop(
                0, nv, ext, (bigv, zi, bigv, zi))
            idx_v[pl.ds(qi * K, 16)] = sel_lo
            idx_v[pl.ds(qi * K + 16, 16)] = sel_hi

            for half, sel in ((0, sel_lo), (1, sel_hi)):
                offs = qi * K * 16 + half * 256 + iota16 * 16
                plsc.store_scatter(nb_v, [offs],
                                   qxv - plsc.load_gather(xx, [sel]))
                plsc.store_scatter(nb_v, [offs + 1],
                                   qyv - plsc.load_gather(xy, [sel]))
                plsc.store_scatter(nb_v, [offs + 2],
                                   qzv - plsc.load_gather(xz, [sel]))

            @pl.when(qi == qchunk - 1)
            def _flush():
                row0 = pl.multiple_of(base + ci * qchunk, 8)
                pltpu.sync_copy(
                    idx_v, idx_hbm.at[pl.ds(row0 * K, qchunk * K)])
                pltpu.sync_copy(
                    nb_v, rel_hbm.at[pl.ds(row0 * K * 16, qchunk * K * 16)])

            return 0

        lax.fori_loop(0, per_w, q_body, 0)

    return k(xyzx, xyzy, xyzz, xbx, xby, xbz)


def _bias_body(rel_ref, w1_ref, b1_ref, w2_ref, b2_ref, out_ref):
    h1 = jnp.dot(rel_ref[...], w1_ref[...],
                 preferred_element_type=jnp.float32) + b1_ref[...]
    h1 = jnp.maximum(h1, 0.0)
    out_ref[...] = jnp.dot(h1, w2_ref[...],
                           preferred_element_type=jnp.float32) + b2_ref[...]


def _bias_mlp(rel16, w1p, b1, w2t, b2, blk):
    rows = rel16.shape[0]
    return pl.pallas_call(
        _bias_body,
        grid=(rows // blk,),
        in_specs=[
            pl.BlockSpec((blk, 16), lambda i: (i, 0)),
            pl.BlockSpec((16, 32), lambda i: (0, 0)),
            pl.BlockSpec((1, 32), lambda i: (0, 0)),
            pl.BlockSpec((32, 16), lambda i: (0, 0)),
            pl.BlockSpec((1, 16), lambda i: (0, 0)),
        ],
        out_specs=pl.BlockSpec((blk, 16), lambda i: (i, 0)),
        out_shape=jax.ShapeDtypeStruct((rows, 16), jnp.float32),
    )(rel16, w1p, b1.reshape(1, 32), w2t, b2.reshape(1, 16))


def _sc_attend(qT, kT, vT, bias3, idx2d, rows):
    nw = 32
    per_w = rows // nw
    scale = float(1.0 / (HD ** 0.5))
    mesh = plsc.VectorSubcoreMesh(core_axis_name="c", subcore_axis_name="s")

    @functools.partial(
        pl.kernel,
        mesh=mesh,
        compiler_params=pltpu.CompilerParams(needs_layout_passes=False),
        out_type=jax.ShapeDtypeStruct((rows, DIM), jnp.float32),
        scratch_types=[
            pltpu.VMEM((per_w * K,), jnp.int32),
            pltpu.VMEM((K, DIM), jnp.float32),
            pltpu.VMEM((K, DIM), jnp.float32),
            pltpu.VMEM((DIM,), jnp.float32),
            pltpu.VMEM((K * H,), jnp.float32),
            pltpu.VMEM((DIM,), jnp.float32),
            pltpu.SemaphoreType.DMA,
            pltpu.SemaphoreType.DMA,
        ],
    )
    def k(q_hbm, k_hbm, v_hbm, bias_hbm, idx_hbm, out_hbm,
          idx_v, kb, vb, qb, sb, ob, semk, semv):
        cid = lax.axis_index("c")
        sid = lax.axis_index("s")
        wid = sid * 2 + cid
        base = pl.multiple_of(wid * per_w, 8)
        pltpu.sync_copy(idx_hbm.at[pl.ds(base * K, per_w * K)], idx_v)

        def attend_one(r, ck, cv):
            pltpu.sync_copy(q_hbm.at[r], qb)
            pltpu.sync_copy(bias_hbm.at[r], sb)
            ck.wait()

            def sc_body(d, accs):
                qv = qb[pl.ds(d * H, H)]
                return tuple(accs[j] + qv * kb[j, pl.ds(d * H, H)]
                             for j in range(K))

            accs = lax.fori_loop(
                0, HD, sc_body,
                tuple(jnp.zeros((H,), jnp.float32) for _ in range(K)))
            s = [accs[j] * scale + sb[pl.ds(j * H, H)] for j in range(K)]

            t = list(s)
            while len(t) > 1:
                t = [jnp.maximum(t[2 * i], t[2 * i + 1])
                     for i in range(len(t) // 2)]
            m = t[0]
            den = jnp.zeros((H,), jnp.float32)
            for j in range(K):
                e = jnp.exp(s[j] - m)
                sb[pl.ds(j * H, H)] = e
                den = den + e
            rden = 1.0 / den

            cv.wait()

            def out_body(j, oaccs):
                a = sb[pl.ds(j * H, H)]
                return tuple(oaccs[d] + a * vb[j, pl.ds(d * H, H)]
                             for d in range(HD))

            oaccs = lax.fori_loop(
                0, K, out_body,
                tuple(jnp.zeros((H,), jnp.float32) for _ in range(HD)))
            for d in range(HD):
                ob[pl.ds(d * H, H)] = oaccs[d] * rden
            pltpu.sync_copy(ob, out_hbm.at[r])

        def q_body(qi, _):
            i0 = pl.multiple_of(qi * K, 8)
            ck = pltpu.async_copy(k_hbm.at[idx_v.at[pl.ds(i0, K)]], kb, semk)
            cv = pltpu.async_copy(v_hbm.at[idx_v.at[pl.ds(i0, K)]], vb, semv)
            attend_one(base + qi, ck, cv)
            return 0

        lax.fori_loop(0, per_w, q_body, 0)

    return k(qT, kT, vT, bias3, idx2d)


def _proj_body(x_ref, w_ref, b_ref, o_ref):
    o_ref[...] = jnp.dot(x_ref[...], w_ref[...],
                         preferred_element_type=jnp.float32) + b_ref[...]


def _out_proj(x2d, w, b, blk):
    n = x2d.shape[0]
    return pl.pallas_call(
        _proj_body,
        grid=(n // blk,),
        in_specs=[
            pl.BlockSpec((blk, DIM), lambda i: (i, 0)),
            pl.BlockSpec((DIM, DIM), lambda i: (0, 0)),
            pl.BlockSpec((1, DIM), lambda i: (0, 0)),
        ],
        out_specs=pl.BlockSpec((blk, DIM), lambda i: (i, 0)),
        out_shape=jax.ShapeDtypeStruct((n, DIM), jnp.float32),
    )(x2d, w, b.reshape(1, DIM))


def kernel(x, xyz, Wq, bq, Wk, bk, Wv, bv, Wo, bo, Wb1, bb1, Wb2, bb2):
    B, N, C = x.shape
    rows = B * N

    a = jnp.arange(DIM)
    p = (a % H) * HD + a // H

    wq_t = Wq[p, :].T
    wk_t = Wk[p, :].T
    wv_t = Wv[p, :].T
    wo_t = Wo[:, p].T

    x2d = x.reshape(rows, DIM)
    qT, kT, vT = _qkv_proj(x2d, wq_t, bq[p], wk_t, bk[p], wv_t, bv[p],
                           blk=256)

    xyz2d = xyz.reshape(rows, 3)
    u = lax.bitcast_convert_type(xyz2d, jnp.uint32)
    u = (u + jnp.uint32(0x7FFF) + ((u >> 16) & jnp.uint32(1))) \
        & jnp.uint32(0xFFFF0000)
    xyzb = lax.bitcast_convert_type(u, jnp.float32)
    idx_flat, rel1 = _sc_knn_rel(xyz2d[:, 0], xyz2d[:, 1], xyz2d[:, 2],
                                 xyzb[:, 0], xyzb[:, 1], xyzb[:, 2],
                                 rows, N)
    rel2 = rel1.reshape(rows * K, 16)

    w1p = jnp.concatenate(
        [Wb1, jnp.zeros((32, 13), jnp.float32)], axis=1).T
    bias2 = _bias_mlp(rel2, w1p, bb1, Wb2.T, bb2, blk=4096)

    outT = _sc_attend(qT, kT, vT, bias2.reshape(rows, K * H),
                      idx_flat, rows)

    y = _out_proj(outT, wo_t, bo, blk=256)
    return y.reshape(B, N, C)


# speedup vs baseline: 8.6756x; 1.0028x over previous
import functools

import jax
import jax.numpy as jnp
from jax import lax
from jax.experimental import pallas as pl
from jax.experimental.pallas import tpu as pltpu
from jax.experimental.pallas import tpu_sc as plsc

DIM = 512
H = 16
K = 32
HD = DIM // H
BIG = 3.0e38


def _qkv_body(x_ref, wq_ref, bq_ref, wk_ref, bk_ref, wv_ref, bv_ref,
              q_ref, k_ref, v_ref):
    xb = x_ref[...]
    q_ref[...] = jnp.dot(xb, wq_ref[...],
                         preferred_element_type=jnp.float32) + bq_ref[...]
    k_ref[...] = jnp.dot(xb, wk_ref[...],
                         preferred_element_type=jnp.float32) + bk_ref[...]
    v_ref[...] = jnp.dot(xb, wv_ref[...],
                         preferred_element_type=jnp.float32) + bv_ref[...]


def _qkv_proj(x2d, wq, bq, wk, bk, wv, bv, blk):
    n = x2d.shape[0]
    bs_w = pl.BlockSpec((DIM, DIM), lambda i: (0, 0))
    bs_b = pl.BlockSpec((1, DIM), lambda i: (0, 0))
    bs_x = pl.BlockSpec((blk, DIM), lambda i: (i, 0))
    out_sd = jax.ShapeDtypeStruct((n, DIM), jnp.float32)
    return pl.pallas_call(
        _qkv_body,
        grid=(n // blk,),
        in_specs=[bs_x, bs_w, bs_b, bs_w, bs_b, bs_w, bs_b],
        out_specs=[bs_x, bs_x, bs_x],
        out_shape=[out_sd, out_sd, out_sd],
    )(x2d, wq, bq.reshape(1, DIM), wk, bk.reshape(1, DIM),
      wv, bv.reshape(1, DIM))


def _sc_knn_rel(xyzx, xyzy, xyzz, xbx, xby, xbz, rows, n):
    nw = 32
    per_w = rows // nw
    qchunk = 64
    ngrp = n // 16
    cap = n + 16
    mesh = plsc.VectorSubcoreMesh(core_axis_name="c", subcore_axis_name="s")

    @functools.partial(
        pl.kernel,
        mesh=mesh,
        compiler_params=pltpu.CompilerParams(needs_layout_passes=False),
        out_type=[
            jax.ShapeDtypeStruct((rows * K,), jnp.int32),
            jax.ShapeDtypeStruct((rows * K * 16,), jnp.float32),
        ],
        scratch_types=[
            pltpu.VMEM((rows,), jnp.float32),
            pltpu.VMEM((rows,), jnp.float32),
            pltpu.VMEM((rows,), jnp.float32),
            pltpu.VMEM((rows,), jnp.float32),
            pltpu.VMEM((rows,), jnp.float32),
            pltpu.VMEM((rows,), jnp.float32),
            pltpu.VMEM((rows,), jnp.float32),
            pltpu.VMEM((n,), jnp.float32),
            pltpu.VMEM((cap,), jnp.float32),
            pltpu.VMEM((cap,), jnp.int32),
            pltpu.VMEM((qchunk * K,), jnp.int32),
            pltpu.VMEM((qchunk * K * 16,), jnp.float32),
        ],
    )
    def k(xx_hbm, xy_hbm, xz_hbm, bx_hbm, by_hbm, bz_hbm, idx_hbm, rel_hbm,
          xx, xy, xz, bx, by, bz, n2, d2r, cval, cidx, idx_v, nb_v):
        cid = lax.axis_index("c")
        sid = lax.axis_index("s")
        wid = sid * 2 + cid
        base = pl.multiple_of(wid * per_w, 8)
        boff = pl.multiple_of((wid // 16) * n, 8)
        pltpu.sync_copy(xx_hbm, xx)
        pltpu.sync_copy(xy_hbm, xy)
        pltpu.sync_copy(xz_hbm, xz)
        pltpu.sync_copy(bx_hbm, bx)
        pltpu.sync_copy(by_hbm, by)
        pltpu.sync_copy(bz_hbm, bz)
        iota16 = lax.iota(jnp.int32, 16)
        zeros16 = jnp.zeros((16,), jnp.float32)
        bigv = jnp.full((16,), BIG, jnp.float32)
        ibig = jnp.full((16,), jnp.int32(2**31 - 1), jnp.int32)

        def n2_body(g, _):
            vx = xx[pl.ds(g * 16, 16)]
            vy = xy[pl.ds(g * 16, 16)]
            vz = xz[pl.ds(g * 16, 16)]
            n2[pl.ds(g * 16, 16)] = vx * vx + vy * vy + vz * vz
            return 0

        lax.fori_loop(0, rows // 16, n2_body, 0)

        def z_body(z, _):
            nb_v[pl.ds(z * 16, 16)] = zeros16
            return 0

        lax.fori_loop(0, qchunk * K, z_body, 0)

        def q_body(qi2, _):
            ci = qi2 // qchunk
            qi = qi2 - ci * qchunk
            r = base + qi2
            rsp = jnp.full((16,), r, jnp.int32)
            qxv = plsc.load_gather(xx, [rsp])
            qyv = plsc.load_gather(xy, [rsp])
            qzv = plsc.load_gather(xz, [rsp])
            qbx = plsc.load_gather(bx, [rsp])
            qby = plsc.load_gather(by, [rsp])
            qbz = plsc.load_gather(bz, [rsp])
            qnv = plsc.load_gather(n2, [rsp])

            def p1(c4, carry):
                m1, m2 = carry
                for u in range(8):
                    c = c4 * 8 + u
                    o = pl.multiple_of(boff + c * 16, 8)
                    s = (bx[pl.ds(o, 16)] * qbx + by[pl.ds(o, 16)] * qby
                         + bz[pl.ds(o, 16)] * qbz)
                    d = jnp.maximum(qnv + n2[pl.ds(o, 16)] - 2.0 * s, 0.0)
                    d2r[pl.ds(c * 16, 16)] = d
                    upd = d < m1
                    m2 = jnp.where(upd, m1, jnp.minimum(m2, d))
                    m1 = jnp.minimum(m1, d)
                return m1, m2

            _, m2 = lax.fori_loop(0, ngrp // 8, p1, (bigv, bigv))
            thr = jnp.full((16,), jnp.max(m2), jnp.float32)

            def p2(c4, off_v):
                for u in range(4):
                    c = c4 * 4 + u
                    v = d2r[pl.ds(c * 16, 16)]
                    msk = v <= thr
                    cs = plsc.cumsum(jnp.where(msk, 1, 0).astype(jnp.int32))
                    pos = off_v + cs - 1
                    plsc.store_scatter(cval, [pos], v, mask=msk)
                    plsc.store_scatter(cidx, [pos], boff + c * 16 + iota16,
                                       mask=msk)
                    pc = plsc.all_reduce_population_count(msk)
                    off_v = off_v + pc
                return off_v

            off_v = lax.fori_loop(0, ngrp // 4, p2,
                                  jnp.zeros((16,), jnp.int32))
            cnt = off_v[0]
            cval[pl.ds(cnt, 16)] = bigv
            nv = (cnt + 15) // 16

            def ext(c2, carry):
                ak, av, bk, bv = carry
                ck = cval[pl.ds(c2 * 16, 16)]
                cv = cidx[pl.ds(c2 * 16, 16)]
                ck, cv = plsc.sort_key_val(ck, cv)
                rk = lax.rev(ck, (0,))
                rv = lax.rev(cv, (0,))
                mm = bk <= rk
                lk = jnp.where(mm, bk, rk)
                lv = jnp.where(mm, bv, rv)
                lk, lv = plsc.sort_key_val(lk, lv)
                rk2 = lax.rev(lk, (0,))
                rv2 = lax.rev(lv, (0,))
                m2 = ak <= rk2
                nak = jnp.where(m2, ak, rk2)
                nav = jnp.where(m2, av, rv2)
                hbk = jnp.where(m2, rk2, ak)
                hbv = jnp.where(m2, rv2, av)
                ak, av = plsc.sort_key_val(nak, nav)
                bk, bv = plsc.sort_key_val(hbk, hbv)
                return ak, av, bk, bv

            zi = jnp.zeros((16,), jnp.int32)
            _, sel_lo, _, sel_hi = lax.fori_loop(
                0, nv, ext, (bigv, zi, bigv, zi))
            idx_v[pl.ds(qi * K, 16)] = sel_lo
            idx_v[pl.ds(qi * K + 16, 16)] = sel_hi

            for half, sel in ((0, sel_lo), (1, sel_hi)):
                offs = qi * K * 16 + half * 256 + iota16 * 16
                plsc.store_scatter(nb_v, [offs],
                                   qxv - plsc.load_gather(xx, [sel]))
                plsc.store_scatter(nb_v, [offs + 1],
                                   qyv - plsc.load_gather(xy, [sel]))
                plsc.store_scatter(nb_v, [offs + 2],
                                   qzv - plsc.load_gather(xz, [sel]))

            @pl.when(qi == qchunk - 1)
            def _flush():
                row0 = pl.multiple_of(base + ci * qchunk, 8)
                pltpu.sync_copy(
                    idx_v, idx_hbm.at[pl.ds(row0 * K, qchunk * K)])
                pltpu.sync_copy(
                    nb_v, rel_hbm.at[pl.ds(row0 * K * 16, qchunk * K * 16)])

            return 0

        lax.fori_loop(0, per_w, q_body, 0)

    return k(xyzx, xyzy, xyzz, xbx, xby, xbz)


def _bias_body(rel_ref, w1_ref, b1_ref, w2_ref, b2_ref, out_ref):
    h1 = jnp.dot(rel_ref[...], w1_ref[...],
                 preferred_element_type=jnp.float32) + b1_ref[...]
    h1 = jnp.maximum(h1, 0.0)
    out_ref[...] = jnp.dot(h1, w2_ref[...],
                           preferred_element_type=jnp.float32) + b2_ref[...]


def _bias_mlp(rel16, w1p, b1, w2t, b2, blk):
    rows = rel16.shape[0]
    return pl.pallas_call(
        _bias_body,
        grid=(rows // blk,),
        in_specs=[
            pl.BlockSpec((blk, 16), lambda i: (i, 0)),
            pl.BlockSpec((16, 32), lambda i: (0, 0)),
            pl.BlockSpec((1, 32), lambda i: (0, 0)),
            pl.BlockSpec((32, 16), lambda i: (0, 0)),
            pl.BlockSpec((1, 16), lambda i: (0, 0)),
        ],
        out_specs=pl.BlockSpec((blk, 16), lambda i: (i, 0)),
        out_shape=jax.ShapeDtypeStruct((rows, 16), jnp.float32),
    )(rel16, w1p, b1.reshape(1, 32), w2t, b2.reshape(1, 16))


def _sc_attend(qT, kT, vT, bias3, idx2d, rows):
    nw = 32
    per_w = rows // nw
    scale = float(1.0 / (HD ** 0.5))
    mesh = plsc.VectorSubcoreMesh(core_axis_name="c", subcore_axis_name="s")

    @functools.partial(
        pl.kernel,
        mesh=mesh,
        compiler_params=pltpu.CompilerParams(needs_layout_passes=False),
        out_type=jax.ShapeDtypeStruct((rows, DIM), jnp.float32),
        scratch_types=[
            pltpu.VMEM((per_w * K,), jnp.int32),
            pltpu.VMEM((K, DIM), jnp.float32),
            pltpu.VMEM((K, DIM), jnp.float32),
            pltpu.VMEM((DIM,), jnp.float32),
            pltpu.VMEM((K * H,), jnp.float32),
            pltpu.VMEM((DIM,), jnp.float32),
            pltpu.SemaphoreType.DMA,
            pltpu.SemaphoreType.DMA,
        ],
    )
    def k(q_hbm, k_hbm, v_hbm, bias_hbm, idx_hbm, out_hbm,
          idx_v, kb, vb, qb, sb, ob, semk, semv):
        cid = lax.axis_index("c")
        sid = lax.axis_index("s")
        wid = sid * 2 + cid
        base = pl.multiple_of(wid * per_w, 8)
        pltpu.sync_copy(idx_hbm.at[pl.ds(base * K, per_w * K)], idx_v)

        def attend_one(r, ck, cv):
            pltpu.sync_copy(q_hbm.at[r], qb)
            pltpu.sync_copy(bias_hbm.at[r], sb)
            ck.wait()

            def sc_body(d, accs):
                qv = qb[pl.ds(d * H, H)]
                return tuple(accs[j] + qv * kb[j, pl.ds(d * H, H)]
                             for j in range(K))

            accs = lax.fori_loop(
                0, HD, sc_body,
                tuple(jnp.zeros((H,), jnp.float32) for _ in range(K)))
            s = [accs[j] * scale + sb[pl.ds(j * H, H)] for j in range(K)]

            t = list(s)
            while len(t) > 1:
                t = [jnp.maximum(t[2 * i], t[2 * i + 1])
                     for i in range(len(t) // 2)]
            m = t[0]
            den = jnp.zeros((H,), jnp.float32)
            for j in range(K):
                e = jnp.exp(s[j] - m)
                sb[pl.ds(j * H, H)] = e
                den = den + e
            rden = 1.0 / den

            cv.wait()

            def out_body(j, oaccs):
                a = sb[pl.ds(j * H, H)]
                return tuple(oaccs[d] + a * vb[j, pl.ds(d * H, H)]
                             for d in range(HD))

            oaccs = lax.fori_loop(
                0, K, out_body,
                tuple(jnp.zeros((H,), jnp.float32) for _ in range(HD)))
            for d in range(HD):
                ob[pl.ds(d * H, H)] = oaccs[d] * rden
            pltpu.sync_copy(ob, out_hbm.at[r])

        def q_body(qi, _):
            i0 = pl.multiple_of(qi * K, 8)
            ck = pltpu.async_copy(k_hbm.at[idx_v.at[pl.ds(i0, K)]], kb, semk)
            cv = pltpu.async_copy(v_hbm.at[idx_v.at[pl.ds(i0, K)]], vb, semv)
            attend_one(base + qi, ck, cv)
            return 0

        lax.fori_loop(0, per_w, q_body, 0)

    return k(qT, kT, vT, bias3, idx2d)


def _proj_body(x_ref, w_ref, b_ref, o_ref):
    o_ref[...] = jnp.dot(x_ref[...], w_ref[...],
                         preferred_element_type=jnp.float32) + b_ref[...]


def _out_proj(x2d, w, b, blk):
    n = x2d.shape[0]
    return pl.pallas_call(
        _proj_body,
        grid=(n // blk,),
        in_specs=[
            pl.BlockSpec((blk, DIM), lambda i: (i, 0)),
            pl.BlockSpec((DIM, DIM), lambda i: (0, 0)),
            pl.BlockSpec((1, DIM), lambda i: (0, 0)),
        ],
        out_specs=pl.BlockSpec((blk, DIM), lambda i: (i, 0)),
        out_shape=jax.ShapeDtypeStruct((n, DIM), jnp.float32),
    )(x2d, w, b.reshape(1, DIM))


def kernel(x, xyz, Wq, bq, Wk, bk, Wv, bv, Wo, bo, Wb1, bb1, Wb2, bb2):
    B, N, C = x.shape
    rows = B * N

    a = jnp.arange(DIM)
    p = (a % H) * HD + a // H

    wq_t = Wq[p, :].T
    wk_t = Wk[p, :].T
    wv_t = Wv[p, :].T
    wo_t = Wo[:, p].T

    x2d = x.reshape(rows, DIM)
    qT, kT, vT = _qkv_proj(x2d, wq_t, bq[p], wk_t, bk[p], wv_t, bv[p],
                           blk=256)

    xyz2d = xyz.reshape(rows, 3)
    u = lax.bitcast_convert_type(xyz2d, jnp.uint32)
    u = (u + jnp.uint32(0x7FFF) + ((u >> 16) & jnp.uint32(1))) \
        & jnp.uint32(0xFFFF0000)
    xyzb = lax.bitcast_convert_type(u, jnp.float32)
    idx_flat, rel1 = _sc_knn_rel(xyz2d[:, 0], xyz2d[:, 1], xyz2d[:, 2],
                                 xyzb[:, 0], xyzb[:, 1], xyzb[:, 2],
                                 rows, N)
    rel2 = rel1.reshape(rows * K, 16)

    w1p = jnp.concatenate(
        [Wb1, jnp.zeros((32, 13), jnp.float32)], axis=1).T
    bias2 = _bias_mlp(rel2, w1p, bb1, Wb2.T, bb2, blk=4096)

    outT = _sc_attend(qT, kT, vT, bias2.reshape(rows, K * H),
                      idx_flat, rows)

    y = _out_proj(outT, wo_t, bo, blk=256)
    return y.reshape(B, N, C)
